# baseline (speedup 1.0000x reference)
import functools

import jax
import jax.numpy as jnp
import numpy as np
from jax import lax
from jax.experimental import pallas as pl
from jax.experimental.pallas import tpu as pltpu
from jax.experimental.pallas import tpu_sc as plsc

_N = 33554432
_NUM_WORKERS = 32
_PER_W = _N // _NUM_WORKERS
_CH = 16384
_NB = 2
_NOUTER = _PER_W // (_CH * _NB)
_LUT_SIZE = 32768

_mesh = plsc.VectorSubcoreMesh(core_axis_name="c", subcore_axis_name="s")


def _build_lut() -> np.ndarray:
    grid = np.linspace(-1.0, 1.0, num=256, endpoint=False).astype(np.float32)[1:]
    g = np.arange(_LUT_SIZE, dtype=np.int64)
    rep_bits = np.where(g < _LUT_SIZE // 2, g << 17, (g << 17) + 1)
    rep = rep_bits.astype(np.uint32).view(np.float32)
    lut = np.digitize(rep, grid).astype(np.int32)
    lut[_LUT_SIZE // 2 - 1] = 128
    return lut


_LUT = _build_lut()


@functools.partial(
    pl.kernel,
    mesh=_mesh,
    out_type=jax.ShapeDtypeStruct((_N,), jnp.int32),
    compiler_params=pltpu.CompilerParams(needs_layout_passes=False),
    scratch_types=[
        pltpu.VMEM((_LUT_SIZE,), jnp.int32),
        pltpu.VMEM((_NB, _CH), jnp.int32),
        pltpu.VMEM((_NB, _CH), jnp.int32),
        pltpu.SemaphoreType.DMA,
        pltpu.SemaphoreType.DMA,
        pltpu.SemaphoreType.DMA,
        pltpu.SemaphoreType.DMA,
    ],
)
def _discretize_sc(x_hbm, lut_hbm, o_hbm, lut_v, in_v, out_v, is0, is1, os0, os1):
    isems = (is0, is1)
    osems = (os0, os1)
    wid = lax.axis_index("s") * 2 + lax.axis_index("c")
    base = wid * _PER_W

    for b in range(_NB):
        pltpu.async_copy(
            x_hbm.at[pl.ds(base + b * _CH, _CH)], in_v.at[b], isems[b]
        )
    pltpu.sync_copy(lut_hbm, lut_v)

    def outer(g2, _):
        for b in range(_NB):
            off = base + (g2 * _NB + b) * _CH
            pltpu.make_async_copy(
                x_hbm.at[pl.ds(off, _CH)], in_v.at[b], isems[b]
            ).wait()

            @pl.when(g2 > 0)
            def _wait_out():
                pltpu.make_async_copy(
                    out_v.at[b], o_hbm.at[pl.ds(off, _CH)], osems[b]
                ).wait()

            @plsc.parallel_loop(0, _CH, step=16, unroll=8)
            def _compute(i):
                bits = in_v[b, pl.ds(i, 16)]
                adj = bits + (bits >> 31)
                idx = (adj >> 17) & 0x7FFF
                out_v[b, pl.ds(i, 16)] = plsc.load_gather(lut_v, [idx])

            pltpu.async_copy(out_v.at[b], o_hbm.at[pl.ds(off, _CH)], osems[b])

            @pl.when(g2 < _NOUTER - 1)
            def _next_in():
                pltpu.async_copy(
                    x_hbm.at[pl.ds(off + _NB * _CH, _CH)], in_v.at[b], isems[b]
                )

        return _

    lax.fori_loop(0, _NOUTER, outer, None)
    for b in range(_NB):
        pltpu.make_async_copy(
            out_v.at[b], o_hbm.at[pl.ds(base, _CH)], osems[b]
        ).wait()


def kernel(actions):
    bits = lax.bitcast_convert_type(actions, jnp.int32)
    return _discretize_sc(bits, jnp.asarray(_LUT))


# speedup vs baseline: 1.4802x; 1.4802x over previous
import functools

import jax
import jax.numpy as jnp
from jax import lax
from jax.experimental import pallas as pl
from jax.experimental.pallas import tpu as pltpu
from jax.experimental.pallas import tpu_sc as plsc

_N = 33554432
_NUM_WORKERS = 32
_CH = 16384
_NB = 2

_M_TC = 20 * 1024 * 1024
_N_SC = _N - _M_TC
_PER_W = _N_SC // _NUM_WORKERS
_NOUTER = _PER_W // (_CH * _NB)
_TC_BLK = 524288
_TC_GRID = _M_TC // _TC_BLK

_mesh = plsc.VectorSubcoreMesh(core_axis_name="c", subcore_axis_name="s")


@functools.partial(
    pl.kernel,
    mesh=_mesh,
    out_type=jax.ShapeDtypeStruct((_N_SC,), jnp.int32),
    scratch_types=[
        pltpu.VMEM((_NB, _CH), jnp.float32),
        pltpu.VMEM((_NB, _CH), jnp.int32),
        pltpu.SemaphoreType.DMA,
        pltpu.SemaphoreType.DMA,
        pltpu.SemaphoreType.DMA,
        pltpu.SemaphoreType.DMA,
    ],
)
def _discretize_sc(x_hbm, o_hbm, in_v, out_v, is0, is1, os0, os1):
    isems = (is0, is1)
    osems = (os0, os1)
    wid = lax.axis_index("s") * 2 + lax.axis_index("c")
    base = _M_TC + wid * _PER_W
    obase = wid * _PER_W

    for b in range(_NB):
        pltpu.async_copy(
            x_hbm.at[pl.ds(base + b * _CH, _CH)], in_v.at[b], isems[b]
        )

    def outer(g2, _):
        for b in range(_NB):
            coff = (g2 * _NB + b) * _CH
            pltpu.make_async_copy(
                x_hbm.at[pl.ds(base + coff, _CH)], in_v.at[b], isems[b]
            ).wait()

            @pl.when(g2 > 0)
            def _wait_out():
                pltpu.make_async_copy(
                    out_v.at[b], o_hbm.at[pl.ds(obase + coff, _CH)], osems[b]
                ).wait()

            @plsc.parallel_loop(0, _CH, step=16, unroll=8)
            def _compute(i):
                x = in_v[b, pl.ds(i, 16)]
                y = x * 128.0
                y = jnp.minimum(jnp.maximum(y, -128.0), 127.0)
                t = y.astype(jnp.int32)
                f = t.astype(jnp.float32)
                out_v[b, pl.ds(i, 16)] = jnp.where(y < f, t - 1, t) + 128

            pltpu.async_copy(
                out_v.at[b], o_hbm.at[pl.ds(obase + coff, _CH)], osems[b]
            )

            @pl.when(g2 < _NOUTER - 1)
            def _next_in():
                pltpu.async_copy(
                    x_hbm.at[pl.ds(base + coff + _NB * _CH, _CH)],
                    in_v.at[b],
                    isems[b],
                )

        return _

    lax.fori_loop(0, _NOUTER, outer, None)
    for b in range(_NB):
        pltpu.make_async_copy(
            out_v.at[b], o_hbm.at[pl.ds(obase, _CH)], osems[b]
        ).wait()


def _tc_body(x_ref, o_ref):
    y = x_ref[...] * 128.0
    y = jnp.minimum(jnp.maximum(y, -128.0), 127.0)
    o_ref[...] = (jnp.floor(y) + 128.0).astype(jnp.int32)


_discretize_tc = pl.pallas_call(
    _tc_body,
    grid=(_TC_GRID,),
    in_specs=[pl.BlockSpec((_TC_BLK,), lambda i: (i,))],
    out_specs=pl.BlockSpec((_TC_BLK,), lambda i: (i,)),
    out_shape=jax.ShapeDtypeStruct((_M_TC,), jnp.int32),
)


def kernel(actions):
    tc_out = _discretize_tc(actions)
    sc_out = _discretize_sc(actions)
    return jnp.concatenate([tc_out, sc_out])
